# baseline (speedup 1.0000x reference)
import functools

import jax
import jax.numpy as jnp
from jax import lax
from jax.experimental import pallas as pl
from jax.experimental.pallas import tpu as pltpu
from jax.experimental.pallas import tpu_sc as plsc

NUM_EXPERTS = 64
HIDDEN = 1024
INTER = 512
TOKENS = 64
TOP_K = 8

EPB = 2
FC = INTER // 2

_PAIRS = TOKENS * TOP_K
_GRID = NUM_EXPERTS * TOKENS


def _combine_sc_body(idx_hbm, w_hbm, out_hbm, idx_v, w_v, acc_v):
    cid = lax.axis_index("c")
    sid = lax.axis_index("s")
    wid = sid * 2 + cid
    pltpu.sync_copy(idx_hbm, idx_v)
    pltpu.sync_copy(w_hbm, w_v)
    e0 = wid * 2
    e1 = e0 + 1
    zero = jnp.zeros((16,), jnp.float32)
    for c in range(TOKENS // 16):
        a0 = zero
        a1 = zero
        for k in range(TOP_K):
            iv = idx_v[pl.ds(k * TOKENS + c * 16, 16)]
            wv = w_v[pl.ds(k * TOKENS + c * 16, 16)]
            a0 = a0 + jnp.where(iv == e0, wv, 0.0)
            a1 = a1 + jnp.where(iv == e1, wv, 0.0)
        acc_v[pl.ds(c * 16, 16)] = a0
        acc_v[pl.ds(TOKENS + c * 16, 16)] = a1
    pltpu.sync_copy(acc_v, out_hbm.at[pl.ds(e0 * TOKENS, 2 * TOKENS)])


def _combine_sc(top_k_index, top_k_weights):
    mesh = plsc.VectorSubcoreMesh(core_axis_name="c", subcore_axis_name="s")
    k = functools.partial(
        pl.kernel,
        mesh=mesh,
        out_type=jax.ShapeDtypeStruct((_GRID,), jnp.float32),
        scratch_types=[
            pltpu.VMEM((_PAIRS,), jnp.int32),
            pltpu.VMEM((_PAIRS,), jnp.float32),
            pltpu.VMEM((2 * TOKENS,), jnp.float32),
        ],
    )(_combine_sc_body)
    flat = k(top_k_index.T.reshape(_PAIRS), top_k_weights.T.reshape(_PAIRS))
    return flat.reshape(NUM_EXPERTS, 1, TOKENS)


def _moe_body(x_ref, cw_ref, g0_ref, g1_ref, u0_ref, u1_ref,
              dn0_ref, dn1_ref, out_ref):
    step = pl.program_id(0)
    x = x_ref[...]
    acc = jnp.zeros((TOKENS, HIDDEN), jnp.float32)
    for i in range(EPB):
        hs = []
        for g_ref, u_ref in ((g0_ref, u0_ref), (g1_ref, u1_ref)):
            gate = jax.lax.dot_general(
                x, g_ref[i], (((1,), (1,)), ((), ())),
                preferred_element_type=jnp.float32)
            up = jax.lax.dot_general(
                x, u_ref[i], (((1,), (1,)), ((), ())),
                preferred_element_type=jnp.float32)
            hs.append(gate * jax.nn.sigmoid(gate) * up)
        h = jnp.concatenate(hs, axis=1)
        out0 = jax.lax.dot_general(
            h, dn0_ref[i], (((1,), (1,)), ((), ())),
            preferred_element_type=jnp.float32)
        out1 = jax.lax.dot_general(
            h, dn1_ref[i], (((1,), (1,)), ((), ())),
            preferred_element_type=jnp.float32)
        out_e = jnp.concatenate([out0, out1], axis=1)
        combine = cw_ref[i, 0]
        acc = acc + out_e * combine[:, None]

    @pl.when(step == 0)
    def _init():
        out_ref[...] = acc

    @pl.when(step > 0)
    def _accum():
        out_ref[...] += acc


def kernel(hidden_states, top_k_index, top_k_weights, gate_up_proj, down_proj):
    combine = _combine_sc(top_k_index, top_k_weights)
    return pl.pallas_call(
        _moe_body,
        grid=(NUM_EXPERTS // EPB,),
        in_specs=[
            pl.BlockSpec((TOKENS, HIDDEN), lambda e: (0, 0)),
            pl.BlockSpec((EPB, 1, TOKENS), lambda e: (e, 0, 0)),
            pl.BlockSpec((EPB, FC, HIDDEN), lambda e: (e, 0, 0)),
            pl.BlockSpec((EPB, FC, HIDDEN), lambda e: (e, 1, 0)),
            pl.BlockSpec((EPB, FC, HIDDEN), lambda e: (e, 2, 0)),
            pl.BlockSpec((EPB, FC, HIDDEN), lambda e: (e, 3, 0)),
            pl.BlockSpec((EPB, HIDDEN // 2, INTER), lambda e: (e, 0, 0)),
            pl.BlockSpec((EPB, HIDDEN // 2, INTER), lambda e: (e, 1, 0)),
        ],
        out_specs=pl.BlockSpec((TOKENS, HIDDEN), lambda e: (0, 0)),
        out_shape=jax.ShapeDtypeStruct((TOKENS, HIDDEN), jnp.float32),
        compiler_params=pltpu.CompilerParams(
            dimension_semantics=("arbitrary",),
        ),
    )(hidden_states, combine,
      gate_up_proj, gate_up_proj, gate_up_proj, gate_up_proj,
      down_proj, down_proj)


# speedup vs baseline: 1.1680x; 1.1680x over previous
import jax
import jax.numpy as jnp
from jax.experimental import pallas as pl
from jax.experimental.pallas import tpu as pltpu

NUM_EXPERTS = 64
HIDDEN = 1024
INTER = 512
TOKENS = 64
TOP_K = 8

EPB = 2
FC = INTER // 2


def _moe_body(x_ref, idx_ref, w_ref, g0_ref, g1_ref, u0_ref, u1_ref,
              dn0_ref, dn1_ref, out_ref):
    step = pl.program_id(0)
    x = x_ref[...]
    acc = jnp.zeros((TOKENS, HIDDEN), jnp.float32)
    for i in range(EPB):
        e = step * EPB + i
        hs = []
        for g_ref, u_ref in ((g0_ref, u0_ref), (g1_ref, u1_ref)):
            gate = jax.lax.dot_general(
                x, g_ref[i], (((1,), (1,)), ((), ())),
                preferred_element_type=jnp.float32)
            up = jax.lax.dot_general(
                x, u_ref[i], (((1,), (1,)), ((), ())),
                preferred_element_type=jnp.float32)
            hs.append(gate * jax.nn.sigmoid(gate) * up)
        h = jnp.concatenate(hs, axis=1)
        out0 = jax.lax.dot_general(
            h, dn0_ref[i], (((1,), (1,)), ((), ())),
            preferred_element_type=jnp.float32)
        out1 = jax.lax.dot_general(
            h, dn1_ref[i], (((1,), (1,)), ((), ())),
            preferred_element_type=jnp.float32)
        out_e = jnp.concatenate([out0, out1], axis=1)
        sel = (idx_ref[...] == e).astype(jnp.float32)
        combine = jnp.sum(sel * w_ref[...], axis=1)
        acc = acc + out_e * combine[:, None]

    @pl.when(step == 0)
    def _init():
        out_ref[...] = acc

    @pl.when(step > 0)
    def _accum():
        out_ref[...] += acc


def kernel(hidden_states, top_k_index, top_k_weights, gate_up_proj, down_proj):
    return pl.pallas_call(
        _moe_body,
        grid=(NUM_EXPERTS // EPB,),
        in_specs=[
            pl.BlockSpec((TOKENS, HIDDEN), lambda e: (0, 0)),
            pl.BlockSpec((TOKENS, TOP_K), lambda e: (0, 0)),
            pl.BlockSpec((TOKENS, TOP_K), lambda e: (0, 0)),
            pl.BlockSpec((EPB, FC, HIDDEN), lambda e: (e, 0, 0)),
            pl.BlockSpec((EPB, FC, HIDDEN), lambda e: (e, 1, 0)),
            pl.BlockSpec((EPB, FC, HIDDEN), lambda e: (e, 2, 0)),
            pl.BlockSpec((EPB, FC, HIDDEN), lambda e: (e, 3, 0)),
            pl.BlockSpec((EPB, HIDDEN // 2, INTER), lambda e: (e, 0, 0)),
            pl.BlockSpec((EPB, HIDDEN // 2, INTER), lambda e: (e, 1, 0)),
        ],
        out_specs=pl.BlockSpec((TOKENS, HIDDEN), lambda e: (0, 0)),
        out_shape=jax.ShapeDtypeStruct((TOKENS, HIDDEN), jnp.float32),
        compiler_params=pltpu.CompilerParams(
            dimension_semantics=("arbitrary",),
        ),
    )(hidden_states, top_k_index, top_k_weights,
      gate_up_proj, gate_up_proj, gate_up_proj, gate_up_proj,
      down_proj, down_proj)
